# baseline (speedup 1.0000x reference)
import functools

import jax
import jax.numpy as jnp
from jax.experimental import pallas as pl
from jax.experimental.pallas import tpu as pltpu

NUM = 32
IMGS_PER_STEP = 2
NUM_OBJS = 24
THRESHOLD = 0.35
NEGPOS_RATIO = 7.0
P_REAL = 16800
P_SUB = 8
P_LANE = 2100
BISECT_ITERS = 34


def _sl1(a, b):
    d = jnp.abs(a - b)
    return jnp.where(d < 1.0, 0.5 * d * d, d - 0.5)


def _topk_sum_scalar(vals, k, n_iters=BISECT_ITERS):
    hi0 = jnp.max(vals) + 1.0
    lo0 = jnp.float32(-2.0)

    def body(_, c):
        lo, hi = c
        mid = 0.5 * (lo + hi)
        cnt = jnp.sum(jnp.where(vals >= mid, 1.0, 0.0))
        ge = cnt >= k
        return jnp.where(ge, mid, lo), jnp.where(ge, hi, mid)

    lo, _ = jax.lax.fori_loop(0, n_iters, body, (lo0, hi0))
    gt = vals > lo
    cnt_gt = jnp.sum(jnp.where(gt, 1.0, 0.0))
    s = jnp.sum(jnp.where(gt, jnp.maximum(vals, 0.0), 0.0))
    return s + (k - cnt_gt) * jnp.maximum(lo, 0.0)


_TX1, _TY1, _TX2, _TY2, _TAREA, _TCX, _TCY, _TW, _TH = range(9)
_TL0 = 9
_TVPACK = 19


def _one_image(u, loc_ref, conf_ref, lm_ref, vis_ref, pri_ref, tgt_ref, iota):
    px1 = pri_ref[0]
    py1 = pri_ref[1]
    px2 = pri_ref[2]
    py2 = pri_ref[3]
    area_p = pri_ref[4]
    pcx = pri_ref[5]
    pcy = pri_ref[6]
    inv10w = pri_ref[7]
    inv10h = pri_ref[8]
    invw = pri_ref[9]
    invh = pri_ref[10]

    bto = jnp.full((P_SUB, P_LANE), -1e30, jnp.float32)
    bti = jnp.zeros((P_SUB, P_LANE), jnp.float32)
    forced = jnp.zeros((P_SUB, P_LANE), jnp.bool_)
    maxj = jnp.full((P_SUB, P_LANE), -1.0, jnp.float32)
    any_valid = jnp.bool_(False)

    bpis = []
    valids = []
    for j in range(NUM_OBJS):
        jf = jnp.float32(j)
        tx1 = tgt_ref[u, j, _TX1]
        ty1 = tgt_ref[u, j, _TY1]
        tx2 = tgt_ref[u, j, _TX2]
        ty2 = tgt_ref[u, j, _TY2]
        area_t = tgt_ref[u, j, _TAREA]
        ix = jnp.maximum(jnp.minimum(tx2, px2) - jnp.maximum(tx1, px1), 0.0)
        iy = jnp.maximum(jnp.minimum(ty2, py2) - jnp.maximum(ty1, py1), 0.0)
        inter = ix * iy
        ov = inter / (area_t + area_p - inter)
        upd = ov > bto
        bti = jnp.where(upd, jf, bti)
        bto = jnp.where(upd, ov, bto)
        m_j = jnp.max(ov)
        valid_j = m_j >= 0.2
        any_valid = jnp.logical_or(any_valid, valid_j)
        iota2 = jnp.where(ov == m_j, iota, jnp.float32(1e30))
        bpis.append(jnp.min(iota2))
        valids.append(valid_j)

    for j in range(NUM_OBJS):
        eq = iota == bpis[j]
        forced = jnp.logical_or(forced, jnp.logical_and(eq, valids[j]))
        maxj = jnp.where(eq, jnp.float32(j), maxj)

    bto = jnp.where(forced, 2.0, bto)
    bti = jnp.where(maxj >= 0.0, maxj, bti)
    pos = jnp.logical_and(bto >= THRESHOLD, any_valid)

    zero = jnp.zeros((P_SUB, P_LANE), jnp.float32)

    mcx, mcy, mw, mh = zero, zero, zero, zero
    for j in range(NUM_OBJS):
        mj = bti == jnp.float32(j)
        mcx = jnp.where(mj, tgt_ref[u, j, _TCX], mcx)
        mcy = jnp.where(mj, tgt_ref[u, j, _TCY], mcy)
        mw = jnp.where(mj, tgt_ref[u, j, _TW], mw)
        mh = jnp.where(mj, tgt_ref[u, j, _TH], mh)
    g0 = (mcx - pcx) * inv10w
    g1 = (mcy - pcy) * inv10h
    g2 = jnp.log(mw * invw) * 5.0
    g3 = jnp.log(mh * invh) * 5.0
    sl_loc = (_sl1(loc_ref[u, 0], g0) + _sl1(loc_ref[u, 1], g1)
              + _sl1(loc_ref[u, 2], g2) + _sl1(loc_ref[u, 3], g3))
    loss_l_i = jnp.sum(jnp.where(pos, sl_loc, 0.0))

    row_sum = zero
    for kpt in range(5):
        lx, ly = zero, zero
        for j in range(NUM_OBJS):
            mj = bti == jnp.float32(j)
            lx = jnp.where(mj, tgt_ref[u, j, _TL0 + 2 * kpt], lx)
            ly = jnp.where(mj, tgt_ref[u, j, _TL0 + 2 * kpt + 1], ly)
        glx = (lx - pcx) * inv10w
        gly = (ly - pcy) * inv10h
        row_sum = row_sum + jnp.where(glx != -1.0,
                                      _sl1(lm_ref[u, 2 * kpt], glx), 0.0)
        row_sum = row_sum + jnp.where(gly != -1.0,
                                      _sl1(lm_ref[u, 2 * kpt + 1], gly), 0.0)
    key = jnp.where(pos, row_sum, -1.0)

    mvp = zero
    for j in range(NUM_OBJS):
        mvp = jnp.where(bti == jnp.float32(j), tgt_ref[u, j, _TVPACK], mvp)
    vbits = mvp.astype(jnp.int32)
    bce_sum = zero
    for kpt in range(5):
        x = vis_ref[u, kpt]
        mv = ((vbits >> kpt) & 1).astype(jnp.float32)
        soft = jnp.log(1.0 + jnp.exp(-jnp.abs(x)))
        logp = jnp.maximum(jnp.minimum(x, 0.0) - soft, -100.0)
        log1mp = jnp.maximum(jnp.minimum(-x, 0.0) - soft, -100.0)
        bce_sum = bce_sum - (mv * logp + (1.0 - mv) * log1mp)
    vis_i = jnp.sum(jnp.where(pos, bce_sum, 0.0))

    c0 = conf_ref[u, 0]
    c1 = conf_ref[u, 1]
    mx = jnp.maximum(c0, c1)
    lse = mx + jnp.log(jnp.exp(c0 - mx) + jnp.exp(c1 - mx))
    xsel = jnp.where(pos, c1, c0)
    ce = lse - xsel
    pos_ce = jnp.sum(jnp.where(pos, ce, 0.0))
    neg_vals = jnp.where(pos, -1.0, ce)
    p_f = jnp.sum(jnp.where(pos, 1.0, 0.0))
    k_i = jnp.minimum(NEGPOS_RATIO * p_f, jnp.float32(P_REAL - 1))
    neg_ce = _topk_sum_scalar(neg_vals, k_i)

    return key, loss_l_i, vis_i, pos_ce + neg_ce, p_f


def _kernel(loc_ref, conf_ref, lm_ref, vis_ref, pri_ref, tgt_ref,
            out_ref, key_ref, accf_ref, acci_ref):
    i = pl.program_id(0)

    @pl.when(i == 0)
    def _init():
        accf_ref[0] = 0.0
        accf_ref[1] = 0.0
        accf_ref[2] = 0.0
        acci_ref[0] = 0

    iota = (jax.lax.broadcasted_iota(jnp.int32, (P_SUB, P_LANE), 0) * P_LANE
            + jax.lax.broadcasted_iota(jnp.int32, (P_SUB, P_LANE), 1)
            ).astype(jnp.float32)

    l_acc = jnp.float32(0.0)
    v_acc = jnp.float32(0.0)
    c_acc = jnp.float32(0.0)
    p_acc = jnp.float32(0.0)
    for u in range(IMGS_PER_STEP):
        key, loss_l_i, vis_i, ce_i, p_f = _one_image(
            u, loc_ref, conf_ref, lm_ref, vis_ref, pri_ref, tgt_ref, iota)
        key_ref[i * IMGS_PER_STEP + u] = key
        l_acc += loss_l_i
        v_acc += vis_i
        c_acc += ce_i
        p_acc += p_f

    accf_ref[0] += l_acc
    accf_ref[1] += v_acc
    accf_ref[2] += c_acc
    acci_ref[0] += p_acc.astype(jnp.int32)

    @pl.when(i == NUM // IMGS_PER_STEP - 1)
    def _finish():
        total_pos = acci_ref[0]
        n_f = jnp.maximum(total_pos.astype(jnp.float32), 1.0)
        size = jnp.maximum(total_pos // 2, 1)
        n2 = size.astype(jnp.float32)
        keys = key_ref[...]
        loss_landm = _topk_sum_scalar(keys, size.astype(jnp.float32))
        out_ref[0] = accf_ref[0] / n_f
        out_ref[1] = accf_ref[2] / n_f
        out_ref[2] = loss_landm / n2
        out_ref[3] = accf_ref[1] / n_f


@functools.partial(jax.jit, static_argnames=("interpret",))
def kernel(loc_data, conf_data, landm_data, visible_data, priors, targets,
           interpret=False):
    num = loc_data.shape[0]

    def cm(x):
        return jnp.transpose(x, (0, 2, 1)).reshape(num, -1, P_SUB, P_LANE)

    pcx, pcy, pw, ph = priors[:, 0], priors[:, 1], priors[:, 2], priors[:, 3]
    px1 = pcx - pw / 2
    py1 = pcy - ph / 2
    px2 = pcx + pw / 2
    py2 = pcy + ph / 2
    area_p = (px2 - px1) * (py2 - py1)
    ptab = jnp.stack([px1, py1, px2, py2, area_p, pcx, pcy,
                      1.0 / (0.1 * pw), 1.0 / (0.1 * ph),
                      1.0 / pw, 1.0 / ph], 0).reshape(11, P_SUB, P_LANE)

    tx1, ty1 = targets[:, :, 0], targets[:, :, 1]
    tx2, ty2 = targets[:, :, 2], targets[:, :, 3]
    vpack = sum(targets[:, :, 16 + k] * (1 << k) for k in range(5))
    ttab = jnp.stack(
        [tx1, ty1, tx2, ty2, (tx2 - tx1) * (ty2 - ty1),
         (tx1 + tx2) / 2, (ty1 + ty2) / 2, tx2 - tx1, ty2 - ty1]
        + [targets[:, :, 4 + k] for k in range(10)] + [vpack], -1)

    g = num // IMGS_PER_STEP
    out = pl.pallas_call(
        _kernel,
        grid=(g,),
        in_specs=[
            pl.BlockSpec((IMGS_PER_STEP, 4, P_SUB, P_LANE),
                         lambda i: (i, 0, 0, 0)),
            pl.BlockSpec((IMGS_PER_STEP, 2, P_SUB, P_LANE),
                         lambda i: (i, 0, 0, 0)),
            pl.BlockSpec((IMGS_PER_STEP, 10, P_SUB, P_LANE),
                         lambda i: (i, 0, 0, 0)),
            pl.BlockSpec((IMGS_PER_STEP, 5, P_SUB, P_LANE),
                         lambda i: (i, 0, 0, 0)),
            pl.BlockSpec((11, P_SUB, P_LANE), lambda i: (0, 0, 0)),
            pl.BlockSpec((IMGS_PER_STEP, NUM_OBJS, 20), lambda i: (i, 0, 0),
                         memory_space=pltpu.SMEM),
        ],
        out_specs=pl.BlockSpec((4,), lambda i: (0,), memory_space=pltpu.SMEM),
        out_shape=jax.ShapeDtypeStruct((4,), jnp.float32),
        scratch_shapes=[
            pltpu.VMEM((NUM, P_SUB, P_LANE), jnp.float32),
            pltpu.SMEM((4,), jnp.float32),
            pltpu.SMEM((2,), jnp.int32),
        ],
        compiler_params=pltpu.CompilerParams(
            dimension_semantics=("arbitrary",),
        ),
        interpret=interpret,
    )(cm(loc_data), cm(conf_data), cm(landm_data), cm(visible_data),
      ptab, ttab)
    return (out[0], out[1], out[2], out[3])


# speedup vs baseline: 1.0007x; 1.0007x over previous
import functools

import jax
import jax.numpy as jnp
from jax.experimental import pallas as pl
from jax.experimental.pallas import tpu as pltpu

NUM = 32
IMGS_PER_STEP = 4
NUM_OBJS = 24
THRESHOLD = 0.35
NEGPOS_RATIO = 7.0
P_REAL = 16800
P_SUB = 8
P_LANE = 2100
BISECT_ITERS = 34


def _sl1(a, b):
    d = jnp.abs(a - b)
    return jnp.where(d < 1.0, 0.5 * d * d, d - 0.5)


def _topk_sum_scalar(vals, k, n_iters=BISECT_ITERS):
    hi0 = jnp.max(vals) + 1.0
    lo0 = jnp.float32(-2.0)

    def body(_, c):
        lo, hi = c
        mid = 0.5 * (lo + hi)
        cnt = jnp.sum(jnp.where(vals >= mid, 1.0, 0.0))
        ge = cnt >= k
        return jnp.where(ge, mid, lo), jnp.where(ge, hi, mid)

    lo, _ = jax.lax.fori_loop(0, n_iters, body, (lo0, hi0))
    gt = vals > lo
    cnt_gt = jnp.sum(jnp.where(gt, 1.0, 0.0))
    s = jnp.sum(jnp.where(gt, jnp.maximum(vals, 0.0), 0.0))
    return s + (k - cnt_gt) * jnp.maximum(lo, 0.0)


_TX1, _TY1, _TX2, _TY2, _TAREA, _TCX, _TCY, _TW, _TH = range(9)
_TL0 = 9
_TVPACK = 19


def _one_image(u, loc_ref, conf_ref, lm_ref, vis_ref, pri_ref, tgt_ref, iota):
    px1 = pri_ref[0]
    py1 = pri_ref[1]
    px2 = pri_ref[2]
    py2 = pri_ref[3]
    area_p = pri_ref[4]
    pcx = pri_ref[5]
    pcy = pri_ref[6]
    inv10w = pri_ref[7]
    inv10h = pri_ref[8]
    invw = pri_ref[9]
    invh = pri_ref[10]

    bto = jnp.full((P_SUB, P_LANE), -1e30, jnp.float32)
    bti = jnp.zeros((P_SUB, P_LANE), jnp.float32)
    forced = jnp.zeros((P_SUB, P_LANE), jnp.bool_)
    maxj = jnp.full((P_SUB, P_LANE), -1.0, jnp.float32)
    any_valid = jnp.bool_(False)

    bpis = []
    valids = []
    for j in range(NUM_OBJS):
        jf = jnp.float32(j)
        tx1 = tgt_ref[u, j, _TX1]
        ty1 = tgt_ref[u, j, _TY1]
        tx2 = tgt_ref[u, j, _TX2]
        ty2 = tgt_ref[u, j, _TY2]
        area_t = tgt_ref[u, j, _TAREA]
        ix = jnp.maximum(jnp.minimum(tx2, px2) - jnp.maximum(tx1, px1), 0.0)
        iy = jnp.maximum(jnp.minimum(ty2, py2) - jnp.maximum(ty1, py1), 0.0)
        inter = ix * iy
        ov = inter / (area_t + area_p - inter)
        upd = ov > bto
        bti = jnp.where(upd, jf, bti)
        bto = jnp.where(upd, ov, bto)
        m_j = jnp.max(ov)
        valid_j = m_j >= 0.2
        any_valid = jnp.logical_or(any_valid, valid_j)
        iota2 = jnp.where(ov == m_j, iota, jnp.float32(1e30))
        bpis.append(jnp.min(iota2))
        valids.append(valid_j)

    for j in range(NUM_OBJS):
        eq = iota == bpis[j]
        forced = jnp.logical_or(forced, jnp.logical_and(eq, valids[j]))
        maxj = jnp.where(eq, jnp.float32(j), maxj)

    bto = jnp.where(forced, 2.0, bto)
    bti = jnp.where(maxj >= 0.0, maxj, bti)
    pos = jnp.logical_and(bto >= THRESHOLD, any_valid)

    zero = jnp.zeros((P_SUB, P_LANE), jnp.float32)

    mcx, mcy, mw, mh, mvp = zero, zero, zero, zero, zero
    for j in range(NUM_OBJS):
        mj = bti == jnp.float32(j)
        mcx = jnp.where(mj, tgt_ref[u, j, _TCX], mcx)
        mcy = jnp.where(mj, tgt_ref[u, j, _TCY], mcy)
        mw = jnp.where(mj, tgt_ref[u, j, _TW], mw)
        mh = jnp.where(mj, tgt_ref[u, j, _TH], mh)
        mvp = jnp.where(mj, tgt_ref[u, j, _TVPACK], mvp)
    g0 = (mcx - pcx) * inv10w
    g1 = (mcy - pcy) * inv10h
    g2 = jnp.log(mw * invw) * 5.0
    g3 = jnp.log(mh * invh) * 5.0
    sl_loc = (_sl1(loc_ref[u, 0], g0) + _sl1(loc_ref[u, 1], g1)
              + _sl1(loc_ref[u, 2], g2) + _sl1(loc_ref[u, 3], g3))
    loss_l_i = jnp.sum(jnp.where(pos, sl_loc, 0.0))

    row_sum = zero
    for kpt in range(5):
        lx, ly = zero, zero
        for j in range(NUM_OBJS):
            mj = bti == jnp.float32(j)
            lx = jnp.where(mj, tgt_ref[u, j, _TL0 + 2 * kpt], lx)
            ly = jnp.where(mj, tgt_ref[u, j, _TL0 + 2 * kpt + 1], ly)
        glx = (lx - pcx) * inv10w
        gly = (ly - pcy) * inv10h
        row_sum = row_sum + jnp.where(glx != -1.0,
                                      _sl1(lm_ref[u, 2 * kpt], glx), 0.0)
        row_sum = row_sum + jnp.where(gly != -1.0,
                                      _sl1(lm_ref[u, 2 * kpt + 1], gly), 0.0)
    key = jnp.where(pos, row_sum, -1.0)

    vbits = mvp.astype(jnp.int32)
    bce_sum = zero
    for kpt in range(5):
        x = vis_ref[u, kpt]
        mv = ((vbits >> kpt) & 1).astype(jnp.float32)
        soft = jnp.log(1.0 + jnp.exp(-jnp.abs(x)))
        logp = jnp.maximum(jnp.minimum(x, 0.0) - soft, -100.0)
        log1mp = jnp.maximum(jnp.minimum(-x, 0.0) - soft, -100.0)
        bce_sum = bce_sum - (mv * logp + (1.0 - mv) * log1mp)
    vis_i = jnp.sum(jnp.where(pos, bce_sum, 0.0))

    c0 = conf_ref[u, 0]
    c1 = conf_ref[u, 1]
    mx = jnp.maximum(c0, c1)
    lse = mx + jnp.log(jnp.exp(c0 - mx) + jnp.exp(c1 - mx))
    xsel = jnp.where(pos, c1, c0)
    ce = lse - xsel
    pos_ce = jnp.sum(jnp.where(pos, ce, 0.0))
    neg_vals = jnp.where(pos, -1.0, ce)
    p_f = jnp.sum(jnp.where(pos, 1.0, 0.0))
    k_i = jnp.minimum(NEGPOS_RATIO * p_f, jnp.float32(P_REAL - 1))
    neg_ce = _topk_sum_scalar(neg_vals, k_i)

    return key, loss_l_i, vis_i, pos_ce + neg_ce, p_f


def _kernel(loc_ref, conf_ref, lm_ref, vis_ref, pri_ref, tgt_ref,
            out_ref, key_ref, accf_ref, acci_ref):
    i = pl.program_id(0)

    @pl.when(i == 0)
    def _init():
        accf_ref[0] = 0.0
        accf_ref[1] = 0.0
        accf_ref[2] = 0.0
        acci_ref[0] = 0

    iota = (jax.lax.broadcasted_iota(jnp.int32, (P_SUB, P_LANE), 0) * P_LANE
            + jax.lax.broadcasted_iota(jnp.int32, (P_SUB, P_LANE), 1)
            ).astype(jnp.float32)

    l_acc = jnp.float32(0.0)
    v_acc = jnp.float32(0.0)
    c_acc = jnp.float32(0.0)
    p_acc = jnp.float32(0.0)
    for u in range(IMGS_PER_STEP):
        key, loss_l_i, vis_i, ce_i, p_f = _one_image(
            u, loc_ref, conf_ref, lm_ref, vis_ref, pri_ref, tgt_ref, iota)
        key_ref[i * IMGS_PER_STEP + u] = key
        l_acc += loss_l_i
        v_acc += vis_i
        c_acc += ce_i
        p_acc += p_f

    accf_ref[0] += l_acc
    accf_ref[1] += v_acc
    accf_ref[2] += c_acc
    acci_ref[0] += p_acc.astype(jnp.int32)

    @pl.when(i == NUM // IMGS_PER_STEP - 1)
    def _finish():
        total_pos = acci_ref[0]
        n_f = jnp.maximum(total_pos.astype(jnp.float32), 1.0)
        size = jnp.maximum(total_pos // 2, 1)
        n2 = size.astype(jnp.float32)
        keys = key_ref[...]
        loss_landm = _topk_sum_scalar(keys, size.astype(jnp.float32))
        out_ref[0] = accf_ref[0] / n_f
        out_ref[1] = accf_ref[2] / n_f
        out_ref[2] = loss_landm / n2
        out_ref[3] = accf_ref[1] / n_f


@functools.partial(jax.jit, static_argnames=("interpret",))
def kernel(loc_data, conf_data, landm_data, visible_data, priors, targets,
           interpret=False):
    num = loc_data.shape[0]

    def cm(x):
        return jnp.transpose(x, (0, 2, 1)).reshape(num, -1, P_SUB, P_LANE)

    pcx, pcy, pw, ph = priors[:, 0], priors[:, 1], priors[:, 2], priors[:, 3]
    px1 = pcx - pw / 2
    py1 = pcy - ph / 2
    px2 = pcx + pw / 2
    py2 = pcy + ph / 2
    area_p = (px2 - px1) * (py2 - py1)
    ptab = jnp.stack([px1, py1, px2, py2, area_p, pcx, pcy,
                      1.0 / (0.1 * pw), 1.0 / (0.1 * ph),
                      1.0 / pw, 1.0 / ph], 0).reshape(11, P_SUB, P_LANE)

    tx1, ty1 = targets[:, :, 0], targets[:, :, 1]
    tx2, ty2 = targets[:, :, 2], targets[:, :, 3]
    vpack = sum(targets[:, :, 16 + k] * (1 << k) for k in range(5))
    ttab = jnp.stack(
        [tx1, ty1, tx2, ty2, (tx2 - tx1) * (ty2 - ty1),
         (tx1 + tx2) / 2, (ty1 + ty2) / 2, tx2 - tx1, ty2 - ty1]
        + [targets[:, :, 4 + k] for k in range(10)] + [vpack], -1)

    g = num // IMGS_PER_STEP
    out = pl.pallas_call(
        _kernel,
        grid=(g,),
        in_specs=[
            pl.BlockSpec((IMGS_PER_STEP, 4, P_SUB, P_LANE),
                         lambda i: (i, 0, 0, 0)),
            pl.BlockSpec((IMGS_PER_STEP, 2, P_SUB, P_LANE),
                         lambda i: (i, 0, 0, 0)),
            pl.BlockSpec((IMGS_PER_STEP, 10, P_SUB, P_LANE),
                         lambda i: (i, 0, 0, 0)),
            pl.BlockSpec((IMGS_PER_STEP, 5, P_SUB, P_LANE),
                         lambda i: (i, 0, 0, 0)),
            pl.BlockSpec((11, P_SUB, P_LANE), lambda i: (0, 0, 0)),
            pl.BlockSpec((IMGS_PER_STEP, NUM_OBJS, 20), lambda i: (i, 0, 0),
                         memory_space=pltpu.SMEM),
        ],
        out_specs=pl.BlockSpec((4,), lambda i: (0,), memory_space=pltpu.SMEM),
        out_shape=jax.ShapeDtypeStruct((4,), jnp.float32),
        scratch_shapes=[
            pltpu.VMEM((NUM, P_SUB, P_LANE), jnp.float32),
            pltpu.SMEM((4,), jnp.float32),
            pltpu.SMEM((2,), jnp.int32),
        ],
        compiler_params=pltpu.CompilerParams(
            dimension_semantics=("arbitrary",),
        ),
        interpret=interpret,
    )(cm(loc_data), cm(conf_data), cm(landm_data), cm(visible_data),
      ptab, ttab)
    return (out[0], out[1], out[2], out[3])


# speedup vs baseline: 1.0023x; 1.0016x over previous
import jax
import jax.numpy as jnp
from jax.experimental import pallas as pl
from jax.experimental.pallas import tpu as pltpu

NUM = 32
IMGS_PER_STEP = 4
NUM_OBJS = 24
THRESHOLD = 0.35
NEGPOS_RATIO = 7.0
P_REAL = 16800
P_SUB = 8
P_LANE = 2100
BISECT_ITERS = 34


def _sl1(a, b):
    d = jnp.abs(a - b)
    return jnp.where(d < 1.0, 0.5 * d * d, d - 0.5)


def _topk_sum_scalar(vals, k, n_iters=BISECT_ITERS):
    hi0 = jnp.max(vals) + 1.0
    lo0 = jnp.float32(-2.0)

    def body(_, c):
        lo, hi = c
        mid = 0.5 * (lo + hi)
        cnt = jnp.sum(jnp.where(vals >= mid, 1.0, 0.0))
        ge = cnt >= k
        return jnp.where(ge, mid, lo), jnp.where(ge, hi, mid)

    lo, _ = jax.lax.fori_loop(0, n_iters, body, (lo0, hi0))
    gt = vals > lo
    cnt_gt = jnp.sum(jnp.where(gt, 1.0, 0.0))
    s = jnp.sum(jnp.where(gt, jnp.maximum(vals, 0.0), 0.0))
    return s + (k - cnt_gt) * jnp.maximum(lo, 0.0)


_TX1, _TY1, _TX2, _TY2, _TAREA, _TCX, _TCY, _TW, _TH = range(9)
_TL0 = 9
_TVPACK = 19


def _one_image(u, loc_ref, conf_ref, lm_ref, vis_ref, pri_ref, tgt_ref, iota):
    px1 = pri_ref[0]
    py1 = pri_ref[1]
    px2 = pri_ref[2]
    py2 = pri_ref[3]
    area_p = pri_ref[4]
    pcx = pri_ref[5]
    pcy = pri_ref[6]
    inv10w = pri_ref[7]
    inv10h = pri_ref[8]
    invw = pri_ref[9]
    invh = pri_ref[10]

    bto = jnp.full((P_SUB, P_LANE), -1e30, jnp.float32)
    bti = jnp.zeros((P_SUB, P_LANE), jnp.float32)
    forced = jnp.zeros((P_SUB, P_LANE), jnp.bool_)
    maxj = jnp.full((P_SUB, P_LANE), -1.0, jnp.float32)
    any_valid = jnp.bool_(False)

    bpis = []
    valids = []
    for j in range(NUM_OBJS):
        jf = jnp.float32(j)
        tx1 = tgt_ref[u, j, _TX1]
        ty1 = tgt_ref[u, j, _TY1]
        tx2 = tgt_ref[u, j, _TX2]
        ty2 = tgt_ref[u, j, _TY2]
        area_t = tgt_ref[u, j, _TAREA]
        ix = jnp.maximum(jnp.minimum(tx2, px2) - jnp.maximum(tx1, px1), 0.0)
        iy = jnp.maximum(jnp.minimum(ty2, py2) - jnp.maximum(ty1, py1), 0.0)
        inter = ix * iy
        ov = inter / (area_t + area_p - inter)
        upd = ov > bto
        bti = jnp.where(upd, jf, bti)
        bto = jnp.where(upd, ov, bto)
        m_j = jnp.max(ov)
        valid_j = m_j >= 0.2
        any_valid = jnp.logical_or(any_valid, valid_j)
        iota2 = jnp.where(ov == m_j, iota, jnp.float32(1e30))
        bpis.append(jnp.min(iota2))
        valids.append(valid_j)

    for j in range(NUM_OBJS):
        eq = iota == bpis[j]
        forced = jnp.logical_or(forced, jnp.logical_and(eq, valids[j]))
        maxj = jnp.where(eq, jnp.float32(j), maxj)

    bto = jnp.where(forced, 2.0, bto)
    bti = jnp.where(maxj >= 0.0, maxj, bti)
    pos = jnp.logical_and(bto >= THRESHOLD, any_valid)

    zero = jnp.zeros((P_SUB, P_LANE), jnp.float32)

    mcx, mcy, mw, mh, mvp = zero, zero, zero, zero, zero
    for j in range(NUM_OBJS):
        mj = bti == jnp.float32(j)
        mcx = jnp.where(mj, tgt_ref[u, j, _TCX], mcx)
        mcy = jnp.where(mj, tgt_ref[u, j, _TCY], mcy)
        mw = jnp.where(mj, tgt_ref[u, j, _TW], mw)
        mh = jnp.where(mj, tgt_ref[u, j, _TH], mh)
        mvp = jnp.where(mj, tgt_ref[u, j, _TVPACK], mvp)
    g0 = (mcx - pcx) * inv10w
    g1 = (mcy - pcy) * inv10h
    g2 = jnp.log(mw * invw) * 5.0
    g3 = jnp.log(mh * invh) * 5.0
    sl_loc = (_sl1(loc_ref[u, 0], g0) + _sl1(loc_ref[u, 1], g1)
              + _sl1(loc_ref[u, 2], g2) + _sl1(loc_ref[u, 3], g3))
    loss_l_i = jnp.sum(jnp.where(pos, sl_loc, 0.0))

    row_sum = zero
    for kpt in range(5):
        lx, ly = zero, zero
        for j in range(NUM_OBJS):
            mj = bti == jnp.float32(j)
            lx = jnp.where(mj, tgt_ref[u, j, _TL0 + 2 * kpt], lx)
            ly = jnp.where(mj, tgt_ref[u, j, _TL0 + 2 * kpt + 1], ly)
        glx = (lx - pcx) * inv10w
        gly = (ly - pcy) * inv10h
        row_sum = row_sum + jnp.where(glx != -1.0,
                                      _sl1(lm_ref[u, 2 * kpt], glx), 0.0)
        row_sum = row_sum + jnp.where(gly != -1.0,
                                      _sl1(lm_ref[u, 2 * kpt + 1], gly), 0.0)
    key = jnp.where(pos, row_sum, -1.0)

    vbits = mvp.astype(jnp.int32)
    bce_sum = zero
    for kpt in range(5):
        x = vis_ref[u, kpt]
        mv = ((vbits >> kpt) & 1).astype(jnp.float32)
        soft = jnp.log(1.0 + jnp.exp(-jnp.abs(x)))
        logp = jnp.maximum(jnp.minimum(x, 0.0) - soft, -100.0)
        log1mp = jnp.maximum(jnp.minimum(-x, 0.0) - soft, -100.0)
        bce_sum = bce_sum - (mv * logp + (1.0 - mv) * log1mp)
    vis_i = jnp.sum(jnp.where(pos, bce_sum, 0.0))

    c0 = conf_ref[u, 0]
    c1 = conf_ref[u, 1]
    mx = jnp.maximum(c0, c1)
    lse = mx + jnp.log(jnp.exp(c0 - mx) + jnp.exp(c1 - mx))
    xsel = jnp.where(pos, c1, c0)
    ce = lse - xsel
    pos_ce = jnp.sum(jnp.where(pos, ce, 0.0))
    neg_vals = jnp.where(pos, -1.0, ce)
    p_f = jnp.sum(jnp.where(pos, 1.0, 0.0))
    k_i = jnp.minimum(NEGPOS_RATIO * p_f, jnp.float32(P_REAL - 1))
    neg_ce = _topk_sum_scalar(neg_vals, k_i)

    return key, loss_l_i, vis_i, pos_ce + neg_ce, p_f


def _kernel(loc_ref, conf_ref, lm_ref, vis_ref, pri_ref, tgt_ref,
            out_ref, key_ref, accf_ref, acci_ref):
    i = pl.program_id(0)

    @pl.when(i == 0)
    def _init():
        accf_ref[0] = 0.0
        accf_ref[1] = 0.0
        accf_ref[2] = 0.0
        acci_ref[0] = 0

    iota = (jax.lax.broadcasted_iota(jnp.int32, (P_SUB, P_LANE), 0) * P_LANE
            + jax.lax.broadcasted_iota(jnp.int32, (P_SUB, P_LANE), 1)
            ).astype(jnp.float32)

    l_acc = jnp.float32(0.0)
    v_acc = jnp.float32(0.0)
    c_acc = jnp.float32(0.0)
    p_acc = jnp.float32(0.0)
    for u in range(IMGS_PER_STEP):
        key, loss_l_i, vis_i, ce_i, p_f = _one_image(
            u, loc_ref, conf_ref, lm_ref, vis_ref, pri_ref, tgt_ref, iota)
        key_ref[i * IMGS_PER_STEP + u] = key
        l_acc += loss_l_i
        v_acc += vis_i
        c_acc += ce_i
        p_acc += p_f

    accf_ref[0] += l_acc
    accf_ref[1] += v_acc
    accf_ref[2] += c_acc
    acci_ref[0] += p_acc.astype(jnp.int32)

    @pl.when(i == NUM // IMGS_PER_STEP - 1)
    def _finish():
        total_pos = acci_ref[0]
        n_f = jnp.maximum(total_pos.astype(jnp.float32), 1.0)
        size = jnp.maximum(total_pos // 2, 1)
        n2 = size.astype(jnp.float32)
        keys = key_ref[...]
        loss_landm = _topk_sum_scalar(keys, size.astype(jnp.float32))
        out_ref[0] = accf_ref[0] / n_f
        out_ref[1] = accf_ref[2] / n_f
        out_ref[2] = loss_landm / n2
        out_ref[3] = accf_ref[1] / n_f


@jax.jit
def kernel(loc_data, conf_data, landm_data, visible_data, priors, targets):
    num = loc_data.shape[0]

    def cm(x):
        return jnp.transpose(x, (0, 2, 1)).reshape(num, -1, P_SUB, P_LANE)

    pcx, pcy, pw, ph = priors[:, 0], priors[:, 1], priors[:, 2], priors[:, 3]
    px1 = pcx - pw / 2
    py1 = pcy - ph / 2
    px2 = pcx + pw / 2
    py2 = pcy + ph / 2
    area_p = (px2 - px1) * (py2 - py1)
    ptab = jnp.stack([px1, py1, px2, py2, area_p, pcx, pcy,
                      1.0 / (0.1 * pw), 1.0 / (0.1 * ph),
                      1.0 / pw, 1.0 / ph], 0).reshape(11, P_SUB, P_LANE)

    tx1, ty1 = targets[:, :, 0], targets[:, :, 1]
    tx2, ty2 = targets[:, :, 2], targets[:, :, 3]
    vpack = sum(targets[:, :, 16 + k] * (1 << k) for k in range(5))
    ttab = jnp.stack(
        [tx1, ty1, tx2, ty2, (tx2 - tx1) * (ty2 - ty1),
         (tx1 + tx2) / 2, (ty1 + ty2) / 2, tx2 - tx1, ty2 - ty1]
        + [targets[:, :, 4 + k] for k in range(10)] + [vpack], -1)

    g = num // IMGS_PER_STEP
    out = pl.pallas_call(
        _kernel,
        grid=(g,),
        in_specs=[
            pl.BlockSpec((IMGS_PER_STEP, 4, P_SUB, P_LANE),
                         lambda i: (i, 0, 0, 0)),
            pl.BlockSpec((IMGS_PER_STEP, 2, P_SUB, P_LANE),
                         lambda i: (i, 0, 0, 0)),
            pl.BlockSpec((IMGS_PER_STEP, 10, P_SUB, P_LANE),
                         lambda i: (i, 0, 0, 0)),
            pl.BlockSpec((IMGS_PER_STEP, 5, P_SUB, P_LANE),
                         lambda i: (i, 0, 0, 0)),
            pl.BlockSpec((11, P_SUB, P_LANE), lambda i: (0, 0, 0)),
            pl.BlockSpec((IMGS_PER_STEP, NUM_OBJS, 20), lambda i: (i, 0, 0),
                         memory_space=pltpu.SMEM),
        ],
        out_specs=pl.BlockSpec((4,), lambda i: (0,), memory_space=pltpu.SMEM),
        out_shape=jax.ShapeDtypeStruct((4,), jnp.float32),
        scratch_shapes=[
            pltpu.VMEM((NUM, P_SUB, P_LANE), jnp.float32),
            pltpu.SMEM((4,), jnp.float32),
            pltpu.SMEM((2,), jnp.int32),
        ],
        compiler_params=pltpu.CompilerParams(
            dimension_semantics=("arbitrary",),
        ),
    )(cm(loc_data), cm(conf_data), cm(landm_data), cm(visible_data),
      ptab, ttab)
    return (out[0], out[1], out[2], out[3])
